# Initial kernel scaffold; baseline (speedup 1.0000x reference)
import jax
import jax.numpy as jnp
from jax.experimental import pallas as pl


def kernel(x, P_w, U1, U2, U3, W1, W2, b2):
    raise NotImplementedError("write your pallas kernel here")


# speedup vs baseline: 1.2881x; 1.2881x over previous
import functools
import math

import jax
import jax.numpy as jnp
from jax import lax
from jax.experimental import pallas as pl
from jax.experimental.pallas import tpu as pltpu

N = 4096
D_IN = 1024
H = 4096
D_OUT = 1024
L = 16
TAU = 1.0

NEG_INF = -1e30


def _top2_gates(z, tau):
    cols = lax.broadcasted_iota(jnp.int32, z.shape, 1)
    v0 = jnp.max(z, axis=1, keepdims=True)
    i0 = jnp.min(jnp.where(z == v0, cols, L), axis=1, keepdims=True)
    m0 = cols == i0
    z1 = jnp.where(m0, NEG_INF, z)
    v1 = jnp.max(z1, axis=1, keepdims=True)
    i1 = jnp.min(jnp.where(z1 == v1, cols, L), axis=1, keepdims=True)
    m1 = cols == i1
    t = jnp.exp((v1 - v0) / (tau + 1e-8))
    w0 = 1.0 / (1.0 + t)
    w1 = t / (1.0 + t)
    return jnp.where(m0, w0, 0.0) + jnp.where(m1, w1, 0.0)


def _router_body(x_ref, pwt_ref, u1t_ref, u2t_ref, u3t_ref,
                 g1_ref, g2_ref, g3_ref):
    xa = jnp.dot(x_ref[...], pwt_ref[...], preferred_element_type=jnp.float32)
    z1 = jnp.dot(xa, u1t_ref[...], preferred_element_type=jnp.float32)
    z2 = jnp.dot(xa, u2t_ref[...], preferred_element_type=jnp.float32)
    z3 = jnp.dot(xa, u3t_ref[...], preferred_element_type=jnp.float32)
    g1_ref[...] = _top2_gates(z1, TAU)
    g2_ref[...] = _top2_gates(z2, TAU)
    g3_ref[...] = _top2_gates(z3, TAU)


def _router(x, P_w, U1, U2, U3):
    bt = 1024
    grid = (N // bt,)
    gate_spec = pl.BlockSpec((bt, L), lambda i: (i, 0))
    return pl.pallas_call(
        _router_body,
        grid=grid,
        in_specs=[
            pl.BlockSpec((bt, D_IN), lambda i: (i, 0)),
            pl.BlockSpec((D_IN, 64), lambda i: (0, 0)),
            pl.BlockSpec((64, L), lambda i: (0, 0)),
            pl.BlockSpec((64, L), lambda i: (0, 0)),
            pl.BlockSpec((64, L), lambda i: (0, 0)),
        ],
        out_specs=[gate_spec, gate_spec, gate_spec],
        out_shape=[jax.ShapeDtypeStruct((N, L), jnp.float32)] * 3,
    )(x, P_w.T, U1.T, U2.T, U3.T)


def _gelu_tanh(v):
    c = math.sqrt(2.0 / math.pi)
    return 0.5 * v * (1.0 + jnp.tanh(c * (v + 0.044715 * (v * v * v))))


def _mix1_body(x_ref, w1_ref, g1t_ref, h_ref, acc_ref):
    e = pl.program_id(1)
    xb = x_ref[...].astype(jnp.bfloat16)
    wb = w1_ref[0].astype(jnp.bfloat16)
    part = lax.dot_general(xb, wb, (((1,), (1,)), ((), ())),
                           preferred_element_type=jnp.float32)
    g = g1t_ref[0, 0, :].reshape(-1, 1)
    part = g * part

    @pl.when(e == 0)
    def _():
        acc_ref[...] = part

    @pl.when(e > 0)
    def _():
        acc_ref[...] = acc_ref[...] + part

    @pl.when(e == L - 1)
    def _():
        h_ref[...] = _gelu_tanh(acc_ref[...]).astype(jnp.bfloat16)


def _mix1(x, W1, G1):
    bh = 512
    grid = (H // bh, L)
    return pl.pallas_call(
        _mix1_body,
        grid=grid,
        in_specs=[
            pl.BlockSpec((N, D_IN), lambda hc, e: (0, 0)),
            pl.BlockSpec((1, bh, D_IN), lambda hc, e: (e, hc, 0)),
            pl.BlockSpec((1, 1, N), lambda hc, e: (e, 0, 0)),
        ],
        out_specs=pl.BlockSpec((N, bh), lambda hc, e: (0, hc)),
        out_shape=jax.ShapeDtypeStruct((N, H), jnp.bfloat16),
        scratch_shapes=[pltpu.VMEM((N, bh), jnp.float32)],
        compiler_params=pltpu.CompilerParams(
            dimension_semantics=("parallel", "arbitrary")),
    )(x, W1, G1.T.reshape(L, 1, N))


def _mix2_body(h_ref, w2_ref, g2t_ref, g3_ref, b2t_ref, y_ref, acc_ref):
    kc = pl.program_id(1)
    e = pl.program_id(2)
    nk = pl.num_programs(1)
    wb = w2_ref[0].astype(jnp.bfloat16)
    part = lax.dot_general(h_ref[...], wb, (((1,), (1,)), ((), ())),
                           preferred_element_type=jnp.float32)
    g = g2t_ref[0, 0, :].reshape(-1, 1)
    part = g * part

    first = jnp.logical_and(kc == 0, e == 0)

    @pl.when(first)
    def _():
        acc_ref[...] = part

    @pl.when(jnp.logical_not(first))
    def _():
        acc_ref[...] = acc_ref[...] + part

    @pl.when(jnp.logical_and(kc == nk - 1, e == L - 1))
    def _():
        bias = lax.dot_general(g3_ref[...], b2t_ref[...],
                               (((1,), (1,)), ((), ())),
                               preferred_element_type=jnp.float32)
        y_ref[...] = acc_ref[...] + bias


def _mix2(h, W2, G2, G3, b2):
    bd = 512
    bk = 1024
    grid = (D_OUT // bd, H // bk, L)
    return pl.pallas_call(
        _mix2_body,
        grid=grid,
        in_specs=[
            pl.BlockSpec((N, bk), lambda dc, kc, e: (0, kc)),
            pl.BlockSpec((1, bd, bk), lambda dc, kc, e: (e, dc, kc)),
            pl.BlockSpec((1, 1, N), lambda dc, kc, e: (e, 0, 0)),
            pl.BlockSpec((N, L), lambda dc, kc, e: (0, 0)),
            pl.BlockSpec((bd, L), lambda dc, kc, e: (dc, 0)),
        ],
        out_specs=pl.BlockSpec((N, bd), lambda dc, kc, e: (0, dc)),
        out_shape=jax.ShapeDtypeStruct((N, D_OUT), jnp.float32),
        scratch_shapes=[pltpu.VMEM((N, bd), jnp.float32)],
        compiler_params=pltpu.CompilerParams(
            dimension_semantics=("parallel", "arbitrary", "arbitrary")),
    )(h, W2, G2.T.reshape(L, 1, N), G3, b2.T)


@jax.jit
def kernel(x, P_w, U1, U2, U3, W1, W2, b2):
    G1, G2, G3 = _router(x, P_w, U1, U2, U3)
    h = _mix1(x, W1, G1)
    return _mix2(h, W2, G2, G3, b2)
